# baseline (speedup 1.0000x reference)
import numpy as np
import jax
import jax.numpy as jnp
from jax.experimental import pallas as pl
from jax.experimental.pallas import tpu as pltpu

B, NLIG, NPOK = 16, 64, 128
NPG = NLIG + NPOK
N = B * NPG
H = 128
NUM_RBF = 16
CUTOFF = 4.0
NUM_LAYERS = 4
NE_CROSS = NLIG * NPOK

_CENTERS = np.linspace(0.0, CUTOFF, NUM_RBF).astype(np.float32)
_GAMMA = 1.0 / max(float(_CENTERS[1] - _CENTERS[0]) ** 2, 1e-06)


def _cpe(n):
    pos = np.arange(n, dtype=np.float32)
    ang = 2.0 * np.pi * pos / max(float(n), 1.0)
    return np.stack([np.sin(ang), np.cos(ang)], -1).astype(np.float32)


_CYCLE = np.concatenate([_cpe(NLIG), _cpe(NPOK)], axis=0)

INV_CNT = np.concatenate([
    np.full((NLIG, 1), 1.0 / (NPOK + 2), np.float32),
    np.full((NPOK, 1), 1.0 / (NLIG + 2), np.float32),
], axis=0)

_R_IN_B, _R_IN_G, _R_IN_BN, _R_EMB0, _R_EMB1, _R_HB1, _R_HB2 = range(7)
_MISC_ROWS = 8


def _silu(x):
    return x * jax.nn.sigmoid(x)


def _ln(x, g, b):
    mu = jnp.mean(x, axis=-1, keepdims=True)
    var = jnp.mean((x - mu) ** 2, axis=-1, keepdims=True)
    return (x - mu) * jax.lax.rsqrt(var + 1e-05) * g + b


def _roll_prev(x):
    return jnp.concatenate([x[-1:], x[:-1]], axis=0)


def _roll_next(x):
    return jnp.concatenate([x[1:], x[:1]], axis=0)


def _fwd_kernel(coords_ref, rad_cross_ref, rad_prev_ref, rad_next_ref,
                cycle_ref, in_w_ref, misc_ref, hw1_ref, hw2_ref,
                w1_ref, w2_ref, uw1_ref, uw2_ref, ee_ref, lmisc_ref,
                out_ref):
    coords = coords_ref[0]
    rad_cross = rad_cross_ref[0]
    rad_prev = rad_prev_ref[0]
    rad_next = rad_next_ref[0]
    cycle = cycle_ref[...]
    in_w = in_w_ref[...]
    misc = misc_ref[...]

    h = (jnp.dot(coords, in_w[0:2], preferred_element_type=jnp.float32)
         + jnp.dot(cycle, in_w[2:4], preferred_element_type=jnp.float32)
         + misc[_R_IN_B:_R_IN_B + 1])
    type_emb = jnp.concatenate([
        jnp.broadcast_to(misc[_R_EMB1:_R_EMB1 + 1], (NLIG, H)),
        jnp.broadcast_to(misc[_R_EMB0:_R_EMB0 + 1], (NPOK, H)),
    ], axis=0)
    h = _ln(h + type_emb, misc[_R_IN_G:_R_IN_G + 1], misc[_R_IN_BN:_R_IN_BN + 1])

    inv_cnt = jnp.concatenate([
        jnp.full((NLIG, 1), 1.0 / (NPOK + 2), jnp.float32),
        jnp.full((NPOK, 1), 1.0 / (NLIG + 2), jnp.float32),
    ], axis=0)

    for l in range(NUM_LAYERS):
        w1 = w1_ref[l]
        w2 = w2_ref[l]
        uw1 = uw1_ref[l]
        uw2 = uw2_ref[l]
        ee = ee_ref[l]
        lm = lmisc_ref[l]
        b1 = lm[0:1]
        b2 = lm[1:2]

        w_src = w1[0:H]
        w_dst = w1[H:2 * H]
        w_ef = w1[2 * H:3 * H]
        w_rad = w1[3 * H:3 * H + 17]

        A = jnp.dot(h, w_src, preferred_element_type=jnp.float32)
        Bv = jnp.dot(h, w_dst, preferred_element_type=jnp.float32)
        consts = jnp.dot(ee, w_ef, preferred_element_type=jnp.float32) + b1

        A_lig, A_pk = A[:NLIG], A[NLIG:]
        Bv_lig, Bv_pk = Bv[:NLIG], Bv[NLIG:]

        r_cross = jnp.dot(rad_cross, w_rad,
                          preferred_element_type=jnp.float32)

        src_pk = jnp.broadcast_to(
            A_pk[:, None, :], (NPOK, NLIG, H)).reshape(NE_CROSS, H)
        dst_lig = jnp.broadcast_to(
            Bv_lig[None, :, :], (NPOK, NLIG, H)).reshape(NE_CROSS, H)
        m = _silu(src_pk + dst_lig + consts[2:3] + r_cross)
        m = _silu(jnp.dot(m, w2, preferred_element_type=jnp.float32) + b2)
        contrib_lig = jnp.sum(m.reshape(NPOK, NLIG, H), axis=0)

        src_lig = jnp.broadcast_to(
            A_lig[None, :, :], (NPOK, NLIG, H)).reshape(NE_CROSS, H)
        dst_pk = jnp.broadcast_to(
            Bv_pk[:, None, :], (NPOK, NLIG, H)).reshape(NE_CROSS, H)
        m = _silu(src_lig + dst_pk + consts[3:4] + r_cross)
        m = _silu(jnp.dot(m, w2, preferred_element_type=jnp.float32) + b2)
        contrib_pk = jnp.sum(m.reshape(NPOK, NLIG, H), axis=1)

        ring_const = jnp.concatenate([
            jnp.broadcast_to(consts[0:1], (NLIG, H)),
            jnp.broadcast_to(consts[1:2], (NPOK, H)),
        ], axis=0)
        A_prev = jnp.concatenate(
            [_roll_prev(A_lig), _roll_prev(A_pk)], axis=0)
        A_next = jnp.concatenate(
            [_roll_next(A_lig), _roll_next(A_pk)], axis=0)
        r_prev = jnp.dot(rad_prev, w_rad, preferred_element_type=jnp.float32)
        r_next = jnp.dot(rad_next, w_rad, preferred_element_type=jnp.float32)
        mp = _silu(A_prev + Bv + ring_const + r_prev)
        mp = _silu(jnp.dot(mp, w2, preferred_element_type=jnp.float32) + b2)
        mn = _silu(A_next + Bv + ring_const + r_next)
        mn = _silu(jnp.dot(mn, w2, preferred_element_type=jnp.float32) + b2)

        cross = jnp.concatenate([contrib_lig, contrib_pk], axis=0)
        agg = (cross + mp + mn) * inv_cnt

        u = _silu(jnp.dot(h, uw1[:H], preferred_element_type=jnp.float32)
                  + jnp.dot(agg, uw1[H:], preferred_element_type=jnp.float32)
                  + lm[2:3])
        u = jnp.dot(u, uw2, preferred_element_type=jnp.float32) + lm[3:4]
        h = _ln(h + u, lm[4:5], lm[5:6])

    pooled = jnp.mean(h[:NLIG], axis=0, keepdims=True)
    t = _silu(jnp.dot(pooled, hw1_ref[...],
                      preferred_element_type=jnp.float32)
              + misc[_R_HB1:_R_HB1 + 1])
    o = jnp.dot(t, hw2_ref[...], preferred_element_type=jnp.float32)
    o = o + misc[_R_HB2:_R_HB2 + 1, 0:1]
    out_ref[...] = jnp.broadcast_to(o, (1, 1, H))


def _rad17(dist):
    rbf = jnp.exp(-_GAMMA * (dist[..., None] - jnp.asarray(_CENTERS)) ** 2)
    return jnp.concatenate([rbf, dist[..., None]], axis=-1)


def kernel(ligand_coords, pocket_coords, params):
    coords = jnp.concatenate([ligand_coords, pocket_coords], axis=1)
    c_lig = coords[:, :NLIG]
    c_pk = coords[:, NLIG:]

    d_cross = jnp.linalg.norm(
        c_pk[:, :, None, :] - c_lig[:, None, :, :], axis=-1)
    rad_cross = _rad17(d_cross.reshape(B, NE_CROSS))

    def ring_d(c):
        return jnp.linalg.norm(jnp.roll(c, -1, axis=1) - c, axis=-1)

    d_lig = ring_d(c_lig)
    d_pk = ring_d(c_pk)
    d_prev = jnp.concatenate(
        [jnp.roll(d_lig, 1, axis=1), jnp.roll(d_pk, 1, axis=1)], axis=1)
    d_next = jnp.concatenate([d_lig, d_pk], axis=1)
    rad_prev = _rad17(d_prev)
    rad_next = _rad17(d_next)

    lp = params['layers']
    w1_all = jnp.stack([l['msg_w1'] for l in lp])
    w2_all = jnp.stack([l['msg_w2'] for l in lp])
    uw1_all = jnp.stack([l['upd_w1'] for l in lp])
    uw2_all = jnp.stack([l['upd_w2'] for l in lp])
    ee_all = jnp.stack([l['edge_emb'] for l in lp])
    lmisc = jnp.stack([
        jnp.stack([l['msg_b1'], l['msg_b2'], l['upd_b1'], l['upd_b2'],
                   l['norm_g'], l['norm_b'],
                   jnp.zeros((H,), jnp.float32), jnp.zeros((H,), jnp.float32)])
        for l in lp])

    hb2 = jnp.broadcast_to(params['head_b2'], (H,))
    misc = jnp.stack([
        params['input_proj_b'], params['input_norm_g'], params['input_norm_b'],
        params['node_type_emb'][0], params['node_type_emb'][1],
        params['head_b1'], hb2, jnp.zeros((H,), jnp.float32)])

    cycle = jnp.asarray(_CYCLE)

    batch_spec = lambda shape: pl.BlockSpec(
        (1,) + shape, lambda b: (b,) + (0,) * len(shape))
    full_spec = lambda shape: pl.BlockSpec(shape, lambda b: (0,) * len(shape))

    out = pl.pallas_call(
        _fwd_kernel,
        grid=(B,),
        in_specs=[
            batch_spec((NPG, 2)),
            batch_spec((NE_CROSS, 17)),
            batch_spec((NPG, 17)),
            batch_spec((NPG, 17)),
            full_spec((NPG, 2)),
            full_spec((4, H)),
            full_spec((_MISC_ROWS, H)),
            full_spec((H, H)),
            full_spec((H, 1)),
            full_spec((NUM_LAYERS, 3 * H + 17, H)),
            full_spec((NUM_LAYERS, H, H)),
            full_spec((NUM_LAYERS, 2 * H, H)),
            full_spec((NUM_LAYERS, H, H)),
            full_spec((NUM_LAYERS, 4, H)),
            full_spec((NUM_LAYERS, 8, H)),
        ],
        out_specs=pl.BlockSpec((1, 1, H), lambda b: (b, 0, 0)),
        out_shape=jax.ShapeDtypeStruct((B, 1, H), jnp.float32),
        compiler_params=pltpu.CompilerParams(
            dimension_semantics=("parallel",)),
    )(coords, rad_cross, rad_prev, rad_next, cycle,
      params['input_proj_w'], misc, params['head_w1'], params['head_w2'],
      w1_all, w2_all, uw1_all, uw2_all, ee_all, lmisc)

    return out[:, 0, :1]


# speedup vs baseline: 1.3658x; 1.3658x over previous
import numpy as np
import jax
import jax.numpy as jnp
from jax.experimental import pallas as pl
from jax.experimental.pallas import tpu as pltpu

B, NLIG, NPOK = 16, 64, 128
NPG = NLIG + NPOK
N = B * NPG
H = 128
NUM_RBF = 16
CUTOFF = 4.0
NUM_LAYERS = 4
NE_CROSS = NLIG * NPOK

_CENTERS = np.linspace(0.0, CUTOFF, NUM_RBF).astype(np.float32)
_GAMMA = 1.0 / max(float(_CENTERS[1] - _CENTERS[0]) ** 2, 1e-06)


def _cpe(n):
    pos = np.arange(n, dtype=np.float32)
    ang = 2.0 * np.pi * pos / max(float(n), 1.0)
    return np.stack([np.sin(ang), np.cos(ang)], -1).astype(np.float32)


_CYCLE = np.concatenate([_cpe(NLIG), _cpe(NPOK)], axis=0)

INV_CNT = np.concatenate([
    np.full((NLIG, 1), 1.0 / (NPOK + 2), np.float32),
    np.full((NPOK, 1), 1.0 / (NLIG + 2), np.float32),
], axis=0)

_R_IN_B, _R_IN_G, _R_IN_BN, _R_EMB0, _R_EMB1, _R_HB1, _R_HB2 = range(7)
_MISC_ROWS = 8

_SUM_J = np.tile(np.eye(NLIG, dtype=np.float32), (1, NPOK))
_SUM_I = np.kron(np.eye(NPOK, dtype=np.float32), np.ones((1, NLIG), np.float32))


def _silu(x):
    return x * (0.5 * jnp.tanh(0.5 * x) + 0.5)


def _silu_h(y):
    return y * jnp.tanh(y) + y


def _ln(x, g, b):
    mu = jnp.mean(x, axis=-1, keepdims=True)
    var = jnp.mean((x - mu) ** 2, axis=-1, keepdims=True)
    return (x - mu) * jax.lax.rsqrt(var + 1e-05) * g + b


def _roll_prev(x):
    return jnp.concatenate([x[-1:], x[:-1]], axis=0)


def _roll_next(x):
    return jnp.concatenate([x[1:], x[:1]], axis=0)


def _fwd_kernel(coords_ref, rad_cross_ref, rad_prev_ref, rad_next_ref,
                cycle_ref, in_w_ref, misc_ref, hw1_ref, hw2_ref,
                w1_ref, w2_ref, uw1_ref, uw2_ref, ee_ref, lmisc_ref,
                sum_j_ref, sum_i_ref,
                out_ref):
    coords = coords_ref[0]
    rad_cross = rad_cross_ref[0]
    rad_prev = rad_prev_ref[0]
    rad_next = rad_next_ref[0]
    cycle = cycle_ref[...]
    in_w = in_w_ref[...]
    misc = misc_ref[...]

    h = (jnp.dot(coords, in_w[0:2], preferred_element_type=jnp.float32)
         + jnp.dot(cycle, in_w[2:4], preferred_element_type=jnp.float32)
         + misc[_R_IN_B:_R_IN_B + 1])
    type_emb = jnp.concatenate([
        jnp.broadcast_to(misc[_R_EMB1:_R_EMB1 + 1], (NLIG, H)),
        jnp.broadcast_to(misc[_R_EMB0:_R_EMB0 + 1], (NPOK, H)),
    ], axis=0)
    h = _ln(h + type_emb, misc[_R_IN_G:_R_IN_G + 1], misc[_R_IN_BN:_R_IN_BN + 1])

    inv_cnt = jnp.concatenate([
        jnp.full((NLIG, 1), 1.0 / (NPOK + 2), jnp.float32),
        jnp.full((NPOK, 1), 1.0 / (NLIG + 2), jnp.float32),
    ], axis=0)

    for l in range(NUM_LAYERS):
        w1 = w1_ref[l]
        w2 = w2_ref[l]
        uw1 = uw1_ref[l]
        uw2 = uw2_ref[l]
        ee = ee_ref[l]
        lm = lmisc_ref[l]
        b2 = lm[1:2]

        w_src = w1[0:H]
        w_dst = w1[H:2 * H]
        w_ef = w1[2 * H:3 * H]
        w_rad = w1[3 * H:3 * H + 17]

        A = jnp.dot(h, w_src, preferred_element_type=jnp.float32)
        Bv = jnp.dot(h, w_dst, preferred_element_type=jnp.float32)
        consts = jnp.dot(ee, w_ef, preferred_element_type=jnp.float32) + lm[0:1]

        A_lig, A_pk = A[:NLIG], A[NLIG:]
        Bv_lig, Bv_pk = Bv[:NLIG], Bv[NLIG:]
        A_pk_c2 = (A_pk + consts[2:3]).astype(jnp.bfloat16)
        A_lig_c3 = (A_lig + consts[3:4]).astype(jnp.bfloat16)
        Bv_lig_b = Bv_lig.astype(jnp.bfloat16)
        Bv_pk_b = Bv_pk.astype(jnp.bfloat16)
        w2b = w2.astype(jnp.bfloat16)
        b2b = b2.astype(jnp.bfloat16)

        r_cross = jnp.dot(rad_cross, w_rad.astype(jnp.bfloat16),
                          preferred_element_type=jnp.float32
                          ).astype(jnp.bfloat16)

        src_pk = jnp.broadcast_to(
            A_pk_c2[:, None, :], (NPOK, NLIG, H)).reshape(NE_CROSS, H)
        dst_lig = jnp.broadcast_to(
            Bv_lig_b[None, :, :], (NPOK, NLIG, H)).reshape(NE_CROSS, H)
        m = _silu_h(src_pk + (dst_lig + r_cross))
        m = _silu_h(jnp.dot(m, w2b, preferred_element_type=jnp.float32
                            ).astype(jnp.bfloat16) + b2b)
        contrib_lig = jnp.dot(sum_j_ref[...], m,
                              preferred_element_type=jnp.float32)

        src_lig = jnp.broadcast_to(
            A_lig_c3[None, :, :], (NPOK, NLIG, H)).reshape(NE_CROSS, H)
        dst_pk = jnp.broadcast_to(
            Bv_pk_b[:, None, :], (NPOK, NLIG, H)).reshape(NE_CROSS, H)
        m = _silu_h(src_lig + (dst_pk + r_cross))
        m = _silu_h(jnp.dot(m, w2b, preferred_element_type=jnp.float32
                            ).astype(jnp.bfloat16) + b2b)
        contrib_pk = jnp.dot(sum_i_ref[...], m,
                             preferred_element_type=jnp.float32)

        ring_const = jnp.concatenate([
            jnp.broadcast_to(consts[0:1], (NLIG, H)),
            jnp.broadcast_to(consts[1:2], (NPOK, H)),
        ], axis=0)
        A_prev = jnp.concatenate(
            [_roll_prev(A_lig), _roll_prev(A_pk)], axis=0) + ring_const
        A_next = jnp.concatenate(
            [_roll_next(A_lig), _roll_next(A_pk)], axis=0) + ring_const
        r_prev = jnp.dot(rad_prev, w_rad, preferred_element_type=jnp.float32)
        r_next = jnp.dot(rad_next, w_rad, preferred_element_type=jnp.float32)
        mp = _silu_h(A_prev + Bv + r_prev)
        mp = _silu_h(jnp.dot(mp, w2, preferred_element_type=jnp.float32) + b2)
        mn = _silu_h(A_next + Bv + r_next)
        mn = _silu_h(jnp.dot(mn, w2, preferred_element_type=jnp.float32) + b2)

        cross = jnp.concatenate([contrib_lig, contrib_pk], axis=0)
        agg = (cross + mp + mn) * inv_cnt

        u = _silu_h(jnp.dot(h, uw1[:H], preferred_element_type=jnp.float32)
                    + jnp.dot(agg, uw1[H:],
                              preferred_element_type=jnp.float32)
                    + lm[2:3])
        u = jnp.dot(u, uw2, preferred_element_type=jnp.float32) + lm[3:4]
        h = _ln(h + u, lm[4:5], lm[5:6])

    pooled = jnp.mean(h[:NLIG], axis=0, keepdims=True)
    t = _silu_h(jnp.dot(pooled, hw1_ref[...],
                        preferred_element_type=jnp.float32)
                + misc[_R_HB1:_R_HB1 + 1])
    o = jnp.dot(t, hw2_ref[...], preferred_element_type=jnp.float32)
    o = o + misc[_R_HB2:_R_HB2 + 1, 0:1]
    out_ref[...] = jnp.broadcast_to(o, (1, 1, H))


def _rad17(dist):
    rbf = jnp.exp(-_GAMMA * (dist[..., None] - jnp.asarray(_CENTERS)) ** 2)
    return jnp.concatenate([rbf, dist[..., None]], axis=-1)


def kernel(ligand_coords, pocket_coords, params):
    coords = jnp.concatenate([ligand_coords, pocket_coords], axis=1)
    c_lig = coords[:, :NLIG]
    c_pk = coords[:, NLIG:]

    d_cross = jnp.linalg.norm(
        c_pk[:, :, None, :] - c_lig[:, None, :, :], axis=-1)
    rad_cross = _rad17(d_cross.reshape(B, NE_CROSS)).astype(
        jnp.bfloat16)

    def ring_d(c):
        return jnp.linalg.norm(jnp.roll(c, -1, axis=1) - c, axis=-1)

    d_lig = ring_d(c_lig)
    d_pk = ring_d(c_pk)
    d_prev = jnp.concatenate(
        [jnp.roll(d_lig, 1, axis=1), jnp.roll(d_pk, 1, axis=1)], axis=1)
    d_next = jnp.concatenate([d_lig, d_pk], axis=1)
    rad_prev = _rad17(d_prev)
    rad_next = _rad17(d_next)

    lp = params['layers']
    w1_all = 0.5 * jnp.stack([l['msg_w1'] for l in lp])
    w2_all = 0.5 * jnp.stack([l['msg_w2'] for l in lp])
    uw1_all = 0.5 * jnp.stack([l['upd_w1'] for l in lp])
    uw2_all = jnp.stack([l['upd_w2'] for l in lp])
    ee_all = jnp.stack([l['edge_emb'] for l in lp])
    lmisc = jnp.stack([
        jnp.stack([0.5 * l['msg_b1'], 0.5 * l['msg_b2'], 0.5 * l['upd_b1'],
                   l['upd_b2'], l['norm_g'], l['norm_b'],
                   jnp.zeros((H,), jnp.float32), jnp.zeros((H,), jnp.float32)])
        for l in lp])

    hb2 = jnp.broadcast_to(params['head_b2'], (H,))
    misc = jnp.stack([
        params['input_proj_b'], params['input_norm_g'], params['input_norm_b'],
        params['node_type_emb'][0], params['node_type_emb'][1],
        0.5 * params['head_b1'], hb2, jnp.zeros((H,), jnp.float32)])

    cycle = jnp.asarray(_CYCLE)

    batch_spec = lambda shape: pl.BlockSpec(
        (1,) + shape, lambda b: (b,) + (0,) * len(shape))
    full_spec = lambda shape: pl.BlockSpec(shape, lambda b: (0,) * len(shape))

    out = pl.pallas_call(
        _fwd_kernel,
        grid=(B,),
        in_specs=[
            batch_spec((NPG, 2)),
            batch_spec((NE_CROSS, 17)),
            batch_spec((NPG, 17)),
            batch_spec((NPG, 17)),
            full_spec((NPG, 2)),
            full_spec((4, H)),
            full_spec((_MISC_ROWS, H)),
            full_spec((H, H)),
            full_spec((H, 1)),
            full_spec((NUM_LAYERS, 3 * H + 17, H)),
            full_spec((NUM_LAYERS, H, H)),
            full_spec((NUM_LAYERS, 2 * H, H)),
            full_spec((NUM_LAYERS, H, H)),
            full_spec((NUM_LAYERS, 4, H)),
            full_spec((NUM_LAYERS, 8, H)),
            full_spec((NLIG, NE_CROSS)),
            full_spec((NPOK, NE_CROSS)),
        ],
        out_specs=pl.BlockSpec((1, 1, H), lambda b: (b, 0, 0)),
        out_shape=jax.ShapeDtypeStruct((B, 1, H), jnp.float32),
        compiler_params=pltpu.CompilerParams(
            dimension_semantics=("parallel",)),
    )(coords, rad_cross, rad_prev, rad_next, cycle,
      params['input_proj_w'], misc, 0.5 * params['head_w1'], params['head_w2'],
      w1_all, w2_all, uw1_all, uw2_all, ee_all, lmisc,
      jnp.asarray(_SUM_J, jnp.bfloat16), jnp.asarray(_SUM_I, jnp.bfloat16))

    return out[:, 0, :1]


# speedup vs baseline: 1.4066x; 1.0299x over previous
import numpy as np
import jax
import jax.numpy as jnp
from jax.experimental import pallas as pl
from jax.experimental.pallas import tpu as pltpu

B, NLIG, NPOK = 16, 64, 128
NPG = NLIG + NPOK
N = B * NPG
H = 128
NUM_RBF = 16
CUTOFF = 4.0
NUM_LAYERS = 4
NE_CROSS = NLIG * NPOK

_CENTERS = np.linspace(0.0, CUTOFF, NUM_RBF).astype(np.float32)
_GAMMA = 1.0 / max(float(_CENTERS[1] - _CENTERS[0]) ** 2, 1e-06)


def _cpe(n):
    pos = np.arange(n, dtype=np.float32)
    ang = 2.0 * np.pi * pos / max(float(n), 1.0)
    return np.stack([np.sin(ang), np.cos(ang)], -1).astype(np.float32)


_CYCLE = np.concatenate([_cpe(NLIG), _cpe(NPOK)], axis=0)

INV_CNT = np.concatenate([
    np.full((NLIG, 1), 1.0 / (NPOK + 2), np.float32),
    np.full((NPOK, 1), 1.0 / (NLIG + 2), np.float32),
], axis=0)

_R_IN_B, _R_IN_G, _R_IN_BN, _R_EMB0, _R_EMB1, _R_HB1, _R_HB2 = range(7)
_MISC_ROWS = 8

_SUM_J = np.tile(np.eye(NLIG, dtype=np.float32), (1, NPOK))
_SUM_I = np.kron(np.eye(NPOK, dtype=np.float32), np.ones((1, NLIG), np.float32))


def _silu(x):
    return x * (0.5 * jnp.tanh(0.5 * x) + 0.5)


def _silu_h(y):
    return y * jnp.tanh(y) + y


def _ln(x, g, b):
    mu = jnp.mean(x, axis=-1, keepdims=True)
    var = jnp.mean((x - mu) ** 2, axis=-1, keepdims=True)
    return (x - mu) * jax.lax.rsqrt(var + 1e-05) * g + b


def _roll_prev(x):
    return jnp.concatenate([x[-1:], x[:-1]], axis=0)


def _roll_next(x):
    return jnp.concatenate([x[1:], x[:1]], axis=0)


def _fwd_kernel(coords_ref, rad_cross_ref, rad_prev_ref, rad_next_ref,
                cycle_ref, in_w_ref, misc_ref, hw1_ref, hw2_ref,
                w1_ref, w2_ref, uw1_ref, uw2_ref, ee_ref, lmisc_ref,
                sum_j_ref, sum_i_ref,
                out_ref):
    coords = coords_ref[0]
    rad_cross = rad_cross_ref[0]
    rad_prev = rad_prev_ref[0]
    rad_next = rad_next_ref[0]
    cycle = cycle_ref[...]
    in_w = in_w_ref[...]
    misc = misc_ref[...]

    h = (jnp.dot(coords, in_w[0:2], preferred_element_type=jnp.float32)
         + jnp.dot(cycle, in_w[2:4], preferred_element_type=jnp.float32)
         + misc[_R_IN_B:_R_IN_B + 1])
    type_emb = jnp.concatenate([
        jnp.broadcast_to(misc[_R_EMB1:_R_EMB1 + 1], (NLIG, H)),
        jnp.broadcast_to(misc[_R_EMB0:_R_EMB0 + 1], (NPOK, H)),
    ], axis=0)
    h = _ln(h + type_emb, misc[_R_IN_G:_R_IN_G + 1], misc[_R_IN_BN:_R_IN_BN + 1])

    inv_cnt = jnp.concatenate([
        jnp.full((NLIG, 1), 1.0 / (NPOK + 2), jnp.float32),
        jnp.full((NPOK, 1), 1.0 / (NLIG + 2), jnp.float32),
    ], axis=0)

    w_rad_all = jnp.concatenate(
        [w1_ref[l][3 * H:3 * H + 17] for l in range(NUM_LAYERS)], axis=1)
    r_prev_all = jnp.dot(rad_prev, w_rad_all,
                         preferred_element_type=jnp.float32
                         ).astype(jnp.bfloat16)
    r_next_all = jnp.dot(rad_next, w_rad_all,
                         preferred_element_type=jnp.float32
                         ).astype(jnp.bfloat16)

    for l in range(NUM_LAYERS):
        h_b = h.astype(jnp.bfloat16)
        w1 = w1_ref[l]
        w2b = w2_ref[l]
        uw1 = uw1_ref[l]
        uw2 = uw2_ref[l]
        ee = ee_ref[l]
        lm = lmisc_ref[l]

        w_src = w1[0:H]
        w_dst = w1[H:2 * H]
        w_ef = w1[2 * H:3 * H]

        A = jnp.dot(h_b, w_src, preferred_element_type=jnp.float32)
        Bv = jnp.dot(h_b, w_dst, preferred_element_type=jnp.float32)
        consts = jnp.dot(ee, w_ef, preferred_element_type=jnp.float32) + lm[0:1]

        A_lig, A_pk = A[:NLIG], A[NLIG:]
        Bv_lig, Bv_pk = Bv[:NLIG], Bv[NLIG:]
        A_pk_c2 = (A_pk + consts[2:3]).astype(jnp.bfloat16)
        A_lig_c3 = (A_lig + consts[3:4]).astype(jnp.bfloat16)
        Bv_lig_b = Bv_lig.astype(jnp.bfloat16)
        Bv_pk_b = Bv_pk.astype(jnp.bfloat16)
        b2b = lm[1:2].astype(jnp.bfloat16)

        r_cross = jnp.dot(rad_cross, w1[3 * H:3 * H + 17],
                          preferred_element_type=jnp.float32
                          ).astype(jnp.bfloat16)

        src_pk = jnp.broadcast_to(
            A_pk_c2[:, None, :], (NPOK, NLIG, H)).reshape(NE_CROSS, H)
        dst_lig = jnp.broadcast_to(
            Bv_lig_b[None, :, :], (NPOK, NLIG, H)).reshape(NE_CROSS, H)
        m = _silu_h(src_pk + (dst_lig + r_cross))
        m = _silu_h(jnp.dot(m, w2b, preferred_element_type=jnp.float32
                            ).astype(jnp.bfloat16) + b2b)
        contrib_lig = jnp.dot(sum_j_ref[...], m,
                              preferred_element_type=jnp.float32)

        src_lig = jnp.broadcast_to(
            A_lig_c3[None, :, :], (NPOK, NLIG, H)).reshape(NE_CROSS, H)
        dst_pk = jnp.broadcast_to(
            Bv_pk_b[:, None, :], (NPOK, NLIG, H)).reshape(NE_CROSS, H)
        m = _silu_h(src_lig + (dst_pk + r_cross))
        m = _silu_h(jnp.dot(m, w2b, preferred_element_type=jnp.float32
                            ).astype(jnp.bfloat16) + b2b)
        contrib_pk = jnp.dot(sum_i_ref[...], m,
                             preferred_element_type=jnp.float32)

        ring_const = jnp.concatenate([
            jnp.broadcast_to(consts[0:1], (NLIG, H)),
            jnp.broadcast_to(consts[1:2], (NPOK, H)),
        ], axis=0)
        A_prev = (jnp.concatenate(
            [_roll_prev(A_lig), _roll_prev(A_pk)], axis=0)
            + ring_const).astype(jnp.bfloat16)
        A_next = (jnp.concatenate(
            [_roll_next(A_lig), _roll_next(A_pk)], axis=0)
            + ring_const).astype(jnp.bfloat16)
        Bv_b = Bv.astype(jnp.bfloat16)
        r_prev = r_prev_all[:, l * H:(l + 1) * H]
        r_next = r_next_all[:, l * H:(l + 1) * H]
        mp = _silu_h(A_prev + Bv_b + r_prev)
        mp = _silu_h(jnp.dot(mp, w2b, preferred_element_type=jnp.float32
                             ).astype(jnp.bfloat16) + b2b)
        mn = _silu_h(A_next + Bv_b + r_next)
        mn = _silu_h(jnp.dot(mn, w2b, preferred_element_type=jnp.float32
                             ).astype(jnp.bfloat16) + b2b)

        cross = jnp.concatenate([contrib_lig, contrib_pk], axis=0)
        agg = (cross + (mp + mn).astype(jnp.float32)) * inv_cnt

        u = _silu_h(jnp.dot(h, uw1[:H], preferred_element_type=jnp.float32)
                    + jnp.dot(agg, uw1[H:],
                              preferred_element_type=jnp.float32)
                    + lm[2:3])
        u = jnp.dot(u, uw2, preferred_element_type=jnp.float32) + lm[3:4]
        h = _ln(h + u, lm[4:5], lm[5:6])

    pooled = jnp.mean(h[:NLIG], axis=0, keepdims=True)
    t = _silu_h(jnp.dot(pooled, hw1_ref[...],
                        preferred_element_type=jnp.float32)
                + misc[_R_HB1:_R_HB1 + 1])
    o = jnp.dot(t, hw2_ref[...], preferred_element_type=jnp.float32)
    o = o + misc[_R_HB2:_R_HB2 + 1, 0:1]
    out_ref[...] = jnp.broadcast_to(o, (1, 1, H))


def _rad17(dist):
    rbf = jnp.exp(-_GAMMA * (dist[..., None] - jnp.asarray(_CENTERS)) ** 2)
    return jnp.concatenate([rbf, dist[..., None]], axis=-1)


def kernel(ligand_coords, pocket_coords, params):
    coords = jnp.concatenate([ligand_coords, pocket_coords], axis=1)
    c_lig = coords[:, :NLIG]
    c_pk = coords[:, NLIG:]

    d_cross = jnp.linalg.norm(
        c_pk[:, :, None, :] - c_lig[:, None, :, :], axis=-1)
    rad_cross = _rad17(d_cross.reshape(B, NE_CROSS)).astype(
        jnp.bfloat16)

    def ring_d(c):
        return jnp.linalg.norm(jnp.roll(c, -1, axis=1) - c, axis=-1)

    d_lig = ring_d(c_lig)
    d_pk = ring_d(c_pk)
    d_prev = jnp.concatenate(
        [jnp.roll(d_lig, 1, axis=1), jnp.roll(d_pk, 1, axis=1)], axis=1)
    d_next = jnp.concatenate([d_lig, d_pk], axis=1)
    rad_prev = _rad17(d_prev).astype(jnp.bfloat16)
    rad_next = _rad17(d_next).astype(jnp.bfloat16)

    lp = params['layers']
    w1_all = (0.5 * jnp.stack([l['msg_w1'] for l in lp])).astype(
        jnp.bfloat16)
    w2_all = (0.5 * jnp.stack([l['msg_w2'] for l in lp])).astype(
        jnp.bfloat16)
    uw1_all = 0.5 * jnp.stack([l['upd_w1'] for l in lp])
    uw2_all = jnp.stack([l['upd_w2'] for l in lp])
    ee_all = jnp.stack([l['edge_emb'] for l in lp]).astype(jnp.bfloat16)
    lmisc = jnp.stack([
        jnp.stack([0.5 * l['msg_b1'], 0.5 * l['msg_b2'], 0.5 * l['upd_b1'],
                   l['upd_b2'], l['norm_g'], l['norm_b'],
                   jnp.zeros((H,), jnp.float32), jnp.zeros((H,), jnp.float32)])
        for l in lp])

    hb2 = jnp.broadcast_to(params['head_b2'], (H,))
    misc = jnp.stack([
        params['input_proj_b'], params['input_norm_g'], params['input_norm_b'],
        params['node_type_emb'][0], params['node_type_emb'][1],
        0.5 * params['head_b1'], hb2, jnp.zeros((H,), jnp.float32)])

    cycle = jnp.asarray(_CYCLE)

    batch_spec = lambda shape: pl.BlockSpec(
        (1,) + shape, lambda b: (b,) + (0,) * len(shape))
    full_spec = lambda shape: pl.BlockSpec(shape, lambda b: (0,) * len(shape))

    out = pl.pallas_call(
        _fwd_kernel,
        grid=(B,),
        in_specs=[
            batch_spec((NPG, 2)),
            batch_spec((NE_CROSS, 17)),
            batch_spec((NPG, 17)),
            batch_spec((NPG, 17)),
            full_spec((NPG, 2)),
            full_spec((4, H)),
            full_spec((_MISC_ROWS, H)),
            full_spec((H, H)),
            full_spec((H, 1)),
            full_spec((NUM_LAYERS, 3 * H + 17, H)),
            full_spec((NUM_LAYERS, H, H)),
            full_spec((NUM_LAYERS, 2 * H, H)),
            full_spec((NUM_LAYERS, H, H)),
            full_spec((NUM_LAYERS, 4, H)),
            full_spec((NUM_LAYERS, 8, H)),
            full_spec((NLIG, NE_CROSS)),
            full_spec((NPOK, NE_CROSS)),
        ],
        out_specs=pl.BlockSpec((1, 1, H), lambda b: (b, 0, 0)),
        out_shape=jax.ShapeDtypeStruct((B, 1, H), jnp.float32),
        compiler_params=pltpu.CompilerParams(
            dimension_semantics=("parallel",)),
    )(coords, rad_cross, rad_prev, rad_next, cycle,
      params['input_proj_w'], misc, 0.5 * params['head_w1'], params['head_w2'],
      w1_all, w2_all, uw1_all, uw2_all, ee_all, lmisc,
      jnp.asarray(_SUM_J, jnp.bfloat16), jnp.asarray(_SUM_I, jnp.bfloat16))

    return out[:, 0, :1]
